# baseline (speedup 1.0000x reference)
import jax
import jax.numpy as jnp
from jax.experimental import pallas as pl
from jax.experimental.pallas import tpu as pltpu

T = 32768
D = 128
B = 16
BT = 8192
NB = T // BT

_C00 = (((0,), (0,)), ((), ()))


def _masks(scope_ref, j):
    t = jax.lax.broadcasted_iota(jnp.int32, (1, BT), 1) + j * BT
    starts = scope_ref[0]
    ends = scope_ref[1]
    cmp_lo = (starts <= t).astype(jnp.float32)
    cmp_hi = (ends <= t).astype(jnp.float32)
    maskf = cmp_lo - cmp_hi
    validf = 1.0 - cmp_hi[B - 1:B]
    return maskf, validf


def _main_kernel(scope_ref, x_ref, w_ref, an_ref, bl_ref,
                 attn_s, m_s, den_s, log_s):
    p = pl.program_id(0)
    j = pl.program_id(1)

    @pl.when((p == 0) & (j == 0))
    def _init():
        m_s[...] = jnp.full((B, 128), -jnp.inf, jnp.float32)
        den_s[...] = jnp.zeros((B, 128), jnp.float32)
        log_s[...] = jnp.zeros((B, 128), jnp.float32)

    @pl.when(p == 0)
    def _acc():
        maskf, validf = _masks(scope_ref, j)
        x = x_ref[...]
        attn_row = jax.lax.dot_general(
            w_ref[...], x, (((1,), (1,)), ((), ())),
            preferred_element_type=jnp.float32)
        attn_s[j, :] = attn_row.reshape(BT)
        blkmax = jnp.where(maskf > 0.5, jnp.broadcast_to(attn_row, (B, BT)),
                           -jnp.inf).max(axis=1, keepdims=True)
        m_old = m_s[:, 0:1]
        m_new = jnp.maximum(m_old, blkmax)
        m_clean = jnp.where(jnp.isfinite(m_new), m_new, 0.0)
        r = jnp.where(jnp.isfinite(m_old), jnp.exp(m_old - m_clean), 0.0)
        m_tok = jax.lax.dot_general(m_clean, maskf, _C00,
                                    preferred_element_type=jnp.float32)
        e_row = validf * jnp.exp(attn_row - m_tok)
        ew = maskf * e_row
        den_blk = jax.lax.dot_general(ew, jnp.ones((BT, 1), jnp.float32),
                                      (((1,), (0,)), ((), ())),
                                      preferred_element_type=jnp.float32)
        rb = jnp.broadcast_to(r, (B, 128))
        den_s[...] = den_s[...] * rb + jnp.broadcast_to(den_blk, (B, 128))
        log_s[...] = log_s[...] * rb + jnp.dot(
            ew, x, preferred_element_type=jnp.float32)
        m_s[...] = jnp.broadcast_to(m_new, (B, 128))

    @pl.when(p == 1)
    def _norm():
        maskf, validf = _masks(scope_ref, j)
        attn_row = attn_s[j, :].reshape(1, BT)
        m = m_s[:, 0:1]
        m_clean = jnp.where(jnp.isfinite(m), m, 0.0)
        den = den_s[:, 0:1]
        m_tok = jax.lax.dot_general(m_clean, maskf, _C00,
                                    preferred_element_type=jnp.float32)
        den_tok = jax.lax.dot_general(den, maskf, _C00,
                                      preferred_element_type=jnp.float32)
        e_row = validf * jnp.exp(attn_row - m_tok)
        an = e_row / jnp.where(den_tok > 0, den_tok, 1.0)
        an_ref[...] = an.reshape(1, 1, BT)

        @pl.when(j == NB - 1)
        def _fin():
            den_f = jnp.where(den_s[...] > 0, den_s[...], 1.0)
            bl_ref[...] = log_s[...] / den_f


def kernel(x, scope, attn_w):
    scope = scope.astype(jnp.int32)
    starts = jnp.concatenate([jnp.zeros((1,), jnp.int32), scope[:B - 1]])
    scope_col = jnp.stack([starts, scope]).reshape(2, B, 1)
    w2 = (attn_w * (D ** (-0.5))).reshape(1, D)
    an3d, bag_logits = pl.pallas_call(
        _main_kernel,
        grid=(2, NB),
        in_specs=[
            pl.BlockSpec((2, B, 1), lambda p, j: (0, 0, 0)),
            pl.BlockSpec((BT, D), lambda p, j: (jnp.where(p == 0, j, NB - 1), 0)),
            pl.BlockSpec((1, D), lambda p, j: (0, 0)),
        ],
        out_specs=[
            pl.BlockSpec((1, 1, BT), lambda p, j: (j, 0, 0)),
            pl.BlockSpec((B, 128), lambda p, j: (0, 0)),
        ],
        out_shape=[
            jax.ShapeDtypeStruct((NB, 1, BT), jnp.float32),
            jax.ShapeDtypeStruct((B, D), jnp.float32),
        ],
        scratch_shapes=[
            pltpu.VMEM((NB, BT), jnp.float32),
            pltpu.VMEM((B, 128), jnp.float32),
            pltpu.VMEM((B, 128), jnp.float32),
            pltpu.VMEM((B, 128), jnp.float32),
        ],
        compiler_params=pltpu.CompilerParams(
            dimension_semantics=("arbitrary", "arbitrary")),
    )(scope_col, x, w2)

    return bag_logits, an3d.reshape(T)


# speedup vs baseline: 1.1506x; 1.1506x over previous
import jax
import jax.numpy as jnp
from jax.experimental import pallas as pl
from jax.experimental.pallas import tpu as pltpu

T = 32768
D = 128
B = 16
BT = 8192
NB = T // BT

_C00 = (((0,), (0,)), ((), ()))


def _main_kernel(scope_ref, x_ref, w_ref, an_ref, bl_ref,
                 attn_s, seg_s, m_s, den_s, log_s):
    p = pl.program_id(0)
    j = pl.program_id(1)

    @pl.when((p == 0) & (j == 0))
    def _init():
        m_s[...] = jnp.full((B, 128), -jnp.inf, jnp.float32)
        den_s[...] = jnp.zeros((B, 128), jnp.float32)
        log_s[...] = jnp.zeros((B, 128), jnp.float32)

    row_ids = jax.lax.broadcasted_iota(jnp.int32, (B, BT), 0)

    @pl.when(p == 0)
    def _acc():
        t = jax.lax.broadcasted_iota(jnp.int32, (1, BT), 1) + j * BT
        seg = jnp.zeros((1, BT), jnp.int32)
        for i in range(B):
            seg = seg + (scope_ref[i] <= t).astype(jnp.int32)
        seg_s[j, :] = seg.reshape(BT)
        x = x_ref[...]
        attn_row = jax.lax.dot_general(
            w_ref[...], x, (((1,), (1,)), ((), ())),
            preferred_element_type=jnp.float32)
        attn_s[j, :] = attn_row.reshape(BT)
        maskb = row_ids == seg
        maskf = maskb.astype(jnp.float32)
        blkmax = jnp.where(maskb, jnp.broadcast_to(attn_row, (B, BT)),
                           -jnp.inf).max(axis=1, keepdims=True)
        m_old = m_s[:, 0:1]
        m_new = jnp.maximum(m_old, blkmax)
        m_clean = jnp.where(jnp.isfinite(m_new), m_new, 0.0)
        r = jnp.where(jnp.isfinite(m_old), jnp.exp(m_old - m_clean), 0.0)
        m_tok = jax.lax.dot_general(m_clean, maskf, _C00,
                                    preferred_element_type=jnp.float32)
        e_row = jnp.where(seg < B, jnp.exp(attn_row - m_tok), 0.0)
        ew = maskf * e_row
        den_blk = jax.lax.dot_general(ew, jnp.ones((BT, 1), jnp.float32),
                                      (((1,), (0,)), ((), ())),
                                      preferred_element_type=jnp.float32)
        rb = jnp.broadcast_to(r, (B, 128))
        den_s[...] = den_s[...] * rb + jnp.broadcast_to(den_blk, (B, 128))
        log_s[...] = log_s[...] * rb + jnp.dot(
            ew, x, preferred_element_type=jnp.float32)
        m_s[...] = jnp.broadcast_to(m_new, (B, 128))

    @pl.when(p == 1)
    def _norm():
        seg = seg_s[j, :].reshape(1, BT)
        attn_row = attn_s[j, :].reshape(1, BT)
        maskf = (row_ids == seg).astype(jnp.float32)
        m = m_s[:, 0:1]
        m_clean = jnp.where(jnp.isfinite(m), m, 0.0)
        den = den_s[:, 0:1]
        m_tok = jax.lax.dot_general(m_clean, maskf, _C00,
                                    preferred_element_type=jnp.float32)
        den_tok = jax.lax.dot_general(den, maskf, _C00,
                                      preferred_element_type=jnp.float32)
        e_row = jnp.where(seg < B, jnp.exp(attn_row - m_tok), 0.0)
        an = e_row / jnp.where(den_tok > 0, den_tok, 1.0)
        an_ref[...] = an.reshape(1, 1, BT)

        @pl.when(j == NB - 1)
        def _fin():
            den_f = jnp.where(den_s[...] > 0, den_s[...], 1.0)
            bl_ref[...] = log_s[...] / den_f


def kernel(x, scope, attn_w):
    scope = scope.astype(jnp.int32)
    w2 = (attn_w * (D ** (-0.5))).reshape(1, D)
    an3d, bag_logits = pl.pallas_call(
        _main_kernel,
        grid=(2, NB),
        in_specs=[
            pl.BlockSpec(memory_space=pltpu.SMEM),
            pl.BlockSpec((BT, D), lambda p, j: (jnp.where(p == 0, j, NB - 1), 0)),
            pl.BlockSpec((1, D), lambda p, j: (0, 0)),
        ],
        out_specs=[
            pl.BlockSpec((1, 1, BT), lambda p, j: (j, 0, 0)),
            pl.BlockSpec((B, 128), lambda p, j: (0, 0)),
        ],
        out_shape=[
            jax.ShapeDtypeStruct((NB, 1, BT), jnp.float32),
            jax.ShapeDtypeStruct((B, D), jnp.float32),
        ],
        scratch_shapes=[
            pltpu.VMEM((NB, BT), jnp.float32),
            pltpu.VMEM((NB, BT), jnp.int32),
            pltpu.VMEM((B, 128), jnp.float32),
            pltpu.VMEM((B, 128), jnp.float32),
            pltpu.VMEM((B, 128), jnp.float32),
        ],
        compiler_params=pltpu.CompilerParams(
            dimension_semantics=("arbitrary", "arbitrary")),
    )(scope, x, w2)

    return bag_logits, an3d.reshape(T)
